# baseline (device time: 34614 ns/iter reference)
import jax
import jax.numpy as jnp
from jax import lax
from jax.experimental import pallas as pl
from jax.experimental.pallas import tpu as pltpu

N_DEV = 4
N_LAYERS = 3
N_PEERS = N_DEV - 1
NC = 4


def kernel(x, Win0, Wout0, Win1, Wout1, Win2, Wout2):
    b, d_shard = x.shape
    h_dim = Win0.shape[1]
    hc = h_dim // NC

    def body(x_ref, win0_ref, wout0_ref, win1_ref, wout1_ref, win2_ref,
             wout2_ref, out_ref, win_v, wout_v, send_buf, comm_ref,
             win_sems, wout_sems, send_sems, recv_sems):
        my_pos = lax.axis_index("i")

        win_refs = [win0_ref, win1_ref, win2_ref]
        wout_refs = [wout0_ref, wout1_ref, wout2_ref]

        win_copies = []
        wout_copies = []
        for l in range(N_LAYERS):
            cw = pltpu.make_async_copy(win_refs[l], win_v.at[l], win_sems.at[l])
            co = pltpu.make_async_copy(wout_refs[l], wout_v.at[l], wout_sems.at[l])
            cw.start()
            co.start()
            win_copies.append(cw)
            wout_copies.append(co)

        x_bf = x_ref[:, :].astype(jnp.bfloat16)

        barrier_sem = pltpu.get_barrier_semaphore()
        for off in range(1, N_DEV):
            pl.semaphore_signal(
                barrier_sem, inc=1,
                device_id=((my_pos + off) % N_DEV,),
                device_id_type=pl.DeviceIdType.MESH,
            )
        pl.semaphore_wait(barrier_sem, N_PEERS)

        for l in range(N_LAYERS):
            win_copies[l].wait()
            partials = []
            rdmas = []
            for c in range(NC):
                win_c = win_v[l, :, c * hc:(c + 1) * hc].astype(jnp.bfloat16)
                partial = jnp.dot(x_bf, win_c, preferred_element_type=jnp.float32)
                partials.append(partial)
                send_buf[l, c, :, :] = partial.astype(jnp.bfloat16)
                for off in range(1, N_DEV):
                    rdma = pltpu.make_async_remote_copy(
                        src_ref=send_buf.at[l, c],
                        dst_ref=comm_ref.at[l, c, off - 1],
                        send_sem=send_sems.at[l, c, off - 1],
                        recv_sem=recv_sems.at[l, c, off - 1],
                        device_id=((my_pos + off) % N_DEV,),
                        device_id_type=pl.DeviceIdType.MESH,
                    )
                    rdma.start()
                    rdmas.append(rdma)

            wout_copies[l].wait()
            acc = None
            for c in range(NC):
                wout_c = wout_v[l, c * hc:(c + 1) * hc, :].astype(jnp.bfloat16)
                for j in range(N_PEERS):
                    recv = pltpu.make_async_remote_copy(
                        src_ref=send_buf.at[l, c],
                        dst_ref=comm_ref.at[l, c, j],
                        send_sem=send_sems.at[l, c, j],
                        recv_sem=recv_sems.at[l, c, j],
                        device_id=(my_pos,),
                        device_id_type=pl.DeviceIdType.MESH,
                    )
                    recv.wait_recv()
                h_c = partials[c]
                for j in range(N_PEERS):
                    h_c = h_c + comm_ref[l, c, j].astype(jnp.float32)
                h_c = jnp.maximum(h_c, 0.0).astype(jnp.bfloat16)
                contrib = jnp.dot(h_c, wout_c, preferred_element_type=jnp.float32)
                acc = contrib if acc is None else acc + contrib

            if l == N_LAYERS - 1:
                out_ref[:, :] = acc
            else:
                x_bf = acc.astype(jnp.bfloat16)

            for rdma in rdmas:
                rdma.wait_send()

    return pl.pallas_call(
        body,
        out_shape=jax.ShapeDtypeStruct((b, d_shard), jnp.float32),
        in_specs=[pl.BlockSpec(memory_space=pltpu.VMEM)]
        + [pl.BlockSpec(memory_space=pl.ANY)] * 6,
        out_specs=pl.BlockSpec(memory_space=pltpu.VMEM),
        scratch_shapes=[
            pltpu.VMEM((N_LAYERS, d_shard, h_dim), jnp.float32),
            pltpu.VMEM((N_LAYERS, h_dim, d_shard), jnp.float32),
            pltpu.VMEM((N_LAYERS, NC, b, hc), jnp.bfloat16),
            pltpu.VMEM((N_LAYERS, NC, N_PEERS, b, hc), jnp.bfloat16),
            pltpu.SemaphoreType.DMA((N_LAYERS,)),
            pltpu.SemaphoreType.DMA((N_LAYERS,)),
            pltpu.SemaphoreType.DMA((N_LAYERS, NC, N_PEERS)),
            pltpu.SemaphoreType.DMA((N_LAYERS, NC, N_PEERS)),
        ],
        compiler_params=pltpu.CompilerParams(
            collective_id=0, vmem_limit_bytes=100 * 1024 * 1024
        ),
    )(x, Win0, Wout0, Win1, Wout1, Win2, Wout2)


# device time: 32620 ns/iter; 1.0611x vs baseline; 1.0611x over previous
import jax
import jax.numpy as jnp
from jax import lax
from jax.experimental import pallas as pl
from jax.experimental.pallas import tpu as pltpu

N_DEV = 4
N_LAYERS = 3
NC = 4


def kernel(x, Win0, Wout0, Win1, Wout1, Win2, Wout2):
    b, d_shard = x.shape
    h_dim = Win0.shape[1]
    hc = h_dim // NC

    def body(x_ref, win0_ref, wout0_ref, win1_ref, wout1_ref, win2_ref,
             wout2_ref, out_ref, win_v, wout_v, sbuf1, sbuf2, ex1, ex2,
             win_sems, wout_sems, send_sems, recv_sems):
        my_pos = lax.axis_index("i")
        partner = [jnp.bitwise_xor(my_pos, 1), 3 - my_pos]

        win_refs = [win0_ref, win1_ref, win2_ref]
        wout_refs = [wout0_ref, wout1_ref, wout2_ref]

        win_copies = []
        wout_copies = []
        for l in range(N_LAYERS):
            cw = pltpu.make_async_copy(win_refs[l], win_v.at[l], win_sems.at[l])
            co = pltpu.make_async_copy(wout_refs[l], wout_v.at[l], wout_sems.at[l])
            cw.start()
            co.start()
            win_copies.append(cw)
            wout_copies.append(co)

        x_bf = x_ref[:, :].astype(jnp.bfloat16)

        barrier_sem = pltpu.get_barrier_semaphore()
        for off in range(1, N_DEV):
            pl.semaphore_signal(
                barrier_sem, inc=1,
                device_id=((my_pos + off) % N_DEV,),
                device_id_type=pl.DeviceIdType.MESH,
            )
        pl.semaphore_wait(barrier_sem, N_DEV - 1)

        def exchange(src_buf, dst_buf, l, c, step):
            return pltpu.make_async_remote_copy(
                src_ref=src_buf.at[l, c],
                dst_ref=dst_buf.at[l, c],
                send_sem=send_sems.at[l, c, step],
                recv_sem=recv_sems.at[l, c, step],
                device_id=(partner[step],),
                device_id_type=pl.DeviceIdType.MESH,
            )

        for l in range(N_LAYERS):
            win_copies[l].wait()
            partials = []
            rdmas = []
            for c in range(NC):
                win_c = win_v[l, :, c * hc:(c + 1) * hc].astype(jnp.bfloat16)
                partial = jnp.dot(x_bf, win_c, preferred_element_type=jnp.float32)
                partials.append(partial)
                sbuf1[l, c, :, :] = partial.astype(jnp.bfloat16)
                r = exchange(sbuf1, ex1, l, c, 0)
                r.start()
                rdmas.append(r)

            s1s = []
            for c in range(NC):
                exchange(sbuf1, ex1, l, c, 0).wait_recv()
                s1 = partials[c] + ex1[l, c].astype(jnp.float32)
                s1s.append(s1)
                sbuf2[l, c, :, :] = s1.astype(jnp.bfloat16)
                r = exchange(sbuf2, ex2, l, c, 1)
                r.start()
                rdmas.append(r)

            wout_copies[l].wait()
            acc = None
            for c in range(NC):
                wout_c = wout_v[l, c * hc:(c + 1) * hc, :].astype(jnp.bfloat16)
                exchange(sbuf2, ex2, l, c, 1).wait_recv()
                h_c = s1s[c] + ex2[l, c].astype(jnp.float32)
                h_c = jnp.maximum(h_c, 0.0).astype(jnp.bfloat16)
                contrib = jnp.dot(h_c, wout_c, preferred_element_type=jnp.float32)
                acc = contrib if acc is None else acc + contrib

            if l == N_LAYERS - 1:
                out_ref[:, :] = acc
            else:
                x_bf = acc.astype(jnp.bfloat16)

            for r in rdmas:
                r.wait_send()

    return pl.pallas_call(
        body,
        out_shape=jax.ShapeDtypeStruct((b, d_shard), jnp.float32),
        in_specs=[pl.BlockSpec(memory_space=pltpu.VMEM)]
        + [pl.BlockSpec(memory_space=pl.ANY)] * 6,
        out_specs=pl.BlockSpec(memory_space=pltpu.VMEM),
        scratch_shapes=[
            pltpu.VMEM((N_LAYERS, d_shard, h_dim), jnp.float32),
            pltpu.VMEM((N_LAYERS, h_dim, d_shard), jnp.float32),
            pltpu.VMEM((N_LAYERS, NC, b, hc), jnp.bfloat16),
            pltpu.VMEM((N_LAYERS, NC, b, hc), jnp.bfloat16),
            pltpu.VMEM((N_LAYERS, NC, b, hc), jnp.bfloat16),
            pltpu.VMEM((N_LAYERS, NC, b, hc), jnp.bfloat16),
            pltpu.SemaphoreType.DMA((N_LAYERS,)),
            pltpu.SemaphoreType.DMA((N_LAYERS,)),
            pltpu.SemaphoreType.DMA((N_LAYERS, NC, 2)),
            pltpu.SemaphoreType.DMA((N_LAYERS, NC, 2)),
        ],
        compiler_params=pltpu.CompilerParams(
            collective_id=0, vmem_limit_bytes=100 * 1024 * 1024
        ),
    )(x, Win0, Wout0, Win1, Wout1, Win2, Wout2)


# device time: 30945 ns/iter; 1.1186x vs baseline; 1.0541x over previous
import jax
import jax.numpy as jnp
from jax import lax
from jax.experimental import pallas as pl
from jax.experimental.pallas import tpu as pltpu

N_DEV = 4
N_LAYERS = 3
NC = 4


def kernel(x, Win0, Wout0, Win1, Wout1, Win2, Wout2):
    b, d_shard = x.shape
    h_dim = Win0.shape[1]
    hc = h_dim // NC

    def body(x_ref, win0_ref, wout0_ref, win1_ref, wout1_ref, win2_ref,
             wout2_ref, out_ref, win_v, wout_v, sbuf1, sbuf2, ex1, ex2,
             win_sems, wout_sems, send_sems, recv_sems):
        my_pos = lax.axis_index("i")
        partner = [jnp.bitwise_xor(my_pos, 1), 3 - my_pos]

        win_refs = [win0_ref, win1_ref, win2_ref]
        wout_refs = [wout0_ref, wout1_ref, wout2_ref]

        def win_copy(l, c):
            cs = slice(c * hc, (c + 1) * hc)
            return pltpu.make_async_copy(
                win_refs[l].at[:, cs], win_v.at[l, :, cs], win_sems.at[l, c]
            )

        def wout_copy(l, c):
            rs = slice(c * hc, (c + 1) * hc)
            return pltpu.make_async_copy(
                wout_refs[l].at[rs, :], wout_v.at[l, rs, :], wout_sems.at[l, c]
            )

        for c in range(NC):
            win_copy(0, c).start()

        x_bf = x_ref[:, :].astype(jnp.bfloat16)

        barrier_sem = pltpu.get_barrier_semaphore()
        for off in range(1, N_DEV):
            pl.semaphore_signal(
                barrier_sem, inc=1,
                device_id=((my_pos + off) % N_DEV,),
                device_id_type=pl.DeviceIdType.MESH,
            )
        pl.semaphore_wait(barrier_sem, N_DEV - 1)

        def exchange(src_buf, dst_buf, l, c, step):
            return pltpu.make_async_remote_copy(
                src_ref=src_buf.at[l, c],
                dst_ref=dst_buf.at[l, c],
                send_sem=send_sems.at[l, c, step],
                recv_sem=recv_sems.at[l, c, step],
                device_id=(partner[step],),
                device_id_type=pl.DeviceIdType.MESH,
            )

        for l in range(N_LAYERS):
            for c in range(NC):
                wout_copy(l, c).start()

            partials = []
            rdmas = []
            for c in range(NC):
                win_copy(l, c).wait()
                win_c = win_v[l, :, c * hc:(c + 1) * hc].astype(jnp.bfloat16)
                partial = jnp.dot(x_bf, win_c, preferred_element_type=jnp.float32)
                partials.append(partial)
                sbuf1[l, c, :, :] = partial.astype(jnp.bfloat16)
                r = exchange(sbuf1, ex1, l, c, 0)
                r.start()
                rdmas.append(r)

            if l + 1 < N_LAYERS:
                for c in range(NC):
                    win_copy(l + 1, c).start()

            s1s = []
            for c in range(NC):
                exchange(sbuf1, ex1, l, c, 0).wait_recv()
                s1 = partials[c] + ex1[l, c].astype(jnp.float32)
                s1s.append(s1)
                sbuf2[l, c, :, :] = s1.astype(jnp.bfloat16)
                r = exchange(sbuf2, ex2, l, c, 1)
                r.start()
                rdmas.append(r)

            acc = None
            for c in range(NC):
                wout_copy(l, c).wait()
                wout_c = wout_v[l, c * hc:(c + 1) * hc, :].astype(jnp.bfloat16)
                exchange(sbuf2, ex2, l, c, 1).wait_recv()
                h_c = s1s[c] + ex2[l, c].astype(jnp.float32)
                h_c = jnp.maximum(h_c, 0.0).astype(jnp.bfloat16)
                contrib = jnp.dot(h_c, wout_c, preferred_element_type=jnp.float32)
                acc = contrib if acc is None else acc + contrib

            if l == N_LAYERS - 1:
                out_ref[:, :] = acc
            else:
                x_bf = acc.astype(jnp.bfloat16)

            for r in rdmas:
                r.wait_send()

    return pl.pallas_call(
        body,
        out_shape=jax.ShapeDtypeStruct((b, d_shard), jnp.float32),
        in_specs=[pl.BlockSpec(memory_space=pltpu.VMEM)]
        + [pl.BlockSpec(memory_space=pl.ANY)] * 6,
        out_specs=pl.BlockSpec(memory_space=pltpu.VMEM),
        scratch_shapes=[
            pltpu.VMEM((N_LAYERS, d_shard, h_dim), jnp.float32),
            pltpu.VMEM((N_LAYERS, h_dim, d_shard), jnp.float32),
            pltpu.VMEM((N_LAYERS, NC, b, hc), jnp.bfloat16),
            pltpu.VMEM((N_LAYERS, NC, b, hc), jnp.bfloat16),
            pltpu.VMEM((N_LAYERS, NC, b, hc), jnp.bfloat16),
            pltpu.VMEM((N_LAYERS, NC, b, hc), jnp.bfloat16),
            pltpu.SemaphoreType.DMA((N_LAYERS, NC)),
            pltpu.SemaphoreType.DMA((N_LAYERS, NC)),
            pltpu.SemaphoreType.DMA((N_LAYERS, NC, 2)),
            pltpu.SemaphoreType.DMA((N_LAYERS, NC, 2)),
        ],
        compiler_params=pltpu.CompilerParams(
            collective_id=0, vmem_limit_bytes=100 * 1024 * 1024
        ),
    )(x, Win0, Wout0, Win1, Wout1, Win2, Wout2)


# device time: 30433 ns/iter; 1.1374x vs baseline; 1.0168x over previous
import jax
import jax.numpy as jnp
from jax import lax
from jax.experimental import pallas as pl
from jax.experimental.pallas import tpu as pltpu

N_DEV = 4
N_LAYERS = 3
NC = 8


def kernel(x, Win0, Wout0, Win1, Wout1, Win2, Wout2):
    b, d_shard = x.shape
    h_dim = Win0.shape[1]
    hc = h_dim // NC

    def body(x_ref, win0_ref, wout0_ref, win1_ref, wout1_ref, win2_ref,
             wout2_ref, out_ref, win_v, wout_v, sbuf1, sbuf2, ex1, ex2,
             win_sems, wout_sems, send_sems, recv_sems):
        my_pos = lax.axis_index("i")
        partner = [jnp.bitwise_xor(my_pos, 1), 3 - my_pos]

        win_refs = [win0_ref, win1_ref, win2_ref]
        wout_refs = [wout0_ref, wout1_ref, wout2_ref]

        def win_copy(l, c):
            cs = slice(c * hc, (c + 1) * hc)
            return pltpu.make_async_copy(
                win_refs[l].at[:, cs], win_v.at[l, :, cs], win_sems.at[l, c]
            )

        def wout_copy(l, c):
            rs = slice(c * hc, (c + 1) * hc)
            return pltpu.make_async_copy(
                wout_refs[l].at[rs, :], wout_v.at[l, rs, :], wout_sems.at[l, c]
            )

        for c in range(NC):
            win_copy(0, c).start()

        x_bf = x_ref[:, :].astype(jnp.bfloat16)

        barrier_sem = pltpu.get_barrier_semaphore()
        for off in range(1, N_DEV):
            pl.semaphore_signal(
                barrier_sem, inc=1,
                device_id=((my_pos + off) % N_DEV,),
                device_id_type=pl.DeviceIdType.MESH,
            )
        pl.semaphore_wait(barrier_sem, N_DEV - 1)

        def exchange(src_buf, dst_buf, l, c, step):
            return pltpu.make_async_remote_copy(
                src_ref=src_buf.at[l, c],
                dst_ref=dst_buf.at[l, c],
                send_sem=send_sems.at[l, c, step],
                recv_sem=recv_sems.at[l, c, step],
                device_id=(partner[step],),
                device_id_type=pl.DeviceIdType.MESH,
            )

        for l in range(N_LAYERS):
            for c in range(NC):
                wout_copy(l, c).start()

            rdmas = []
            for c in range(NC):
                win_copy(l, c).wait()
                win_c = win_v[l, :, c * hc:(c + 1) * hc].astype(jnp.bfloat16)
                partial = jnp.dot(x_bf, win_c, preferred_element_type=jnp.float32)
                sbuf1[l, c, :, :] = partial.astype(jnp.bfloat16)
                r = exchange(sbuf1, ex1, l, c, 0)
                r.start()
                rdmas.append(r)

            if l + 1 < N_LAYERS:
                for c in range(NC):
                    win_copy(l + 1, c).start()

            for c in range(NC):
                exchange(sbuf1, ex1, l, c, 0).wait_recv()
                sbuf2[l, c, :, :] = sbuf1[l, c] + ex1[l, c]
                r = exchange(sbuf2, ex2, l, c, 1)
                r.start()
                rdmas.append(r)

            acc = None
            for c in range(NC):
                wout_copy(l, c).wait()
                wout_c = wout_v[l, c * hc:(c + 1) * hc, :].astype(jnp.bfloat16)
                exchange(sbuf2, ex2, l, c, 1).wait_recv()
                h_c = jnp.maximum(sbuf2[l, c] + ex2[l, c], 0.0)
                contrib = jnp.dot(h_c, wout_c, preferred_element_type=jnp.float32)
                acc = contrib if acc is None else acc + contrib

            if l == N_LAYERS - 1:
                out_ref[:, :] = acc
            else:
                x_bf = acc.astype(jnp.bfloat16)

            for r in rdmas:
                r.wait_send()

    return pl.pallas_call(
        body,
        out_shape=jax.ShapeDtypeStruct((b, d_shard), jnp.float32),
        in_specs=[pl.BlockSpec(memory_space=pltpu.VMEM)]
        + [pl.BlockSpec(memory_space=pl.ANY)] * 6,
        out_specs=pl.BlockSpec(memory_space=pltpu.VMEM),
        scratch_shapes=[
            pltpu.VMEM((N_LAYERS, d_shard, h_dim), jnp.float32),
            pltpu.VMEM((N_LAYERS, h_dim, d_shard), jnp.float32),
            pltpu.VMEM((N_LAYERS, NC, b, hc), jnp.bfloat16),
            pltpu.VMEM((N_LAYERS, NC, b, hc), jnp.bfloat16),
            pltpu.VMEM((N_LAYERS, NC, b, hc), jnp.bfloat16),
            pltpu.VMEM((N_LAYERS, NC, b, hc), jnp.bfloat16),
            pltpu.SemaphoreType.DMA((N_LAYERS, NC)),
            pltpu.SemaphoreType.DMA((N_LAYERS, NC)),
            pltpu.SemaphoreType.DMA((N_LAYERS, NC, 2)),
            pltpu.SemaphoreType.DMA((N_LAYERS, NC, 2)),
        ],
        compiler_params=pltpu.CompilerParams(
            collective_id=0, vmem_limit_bytes=100 * 1024 * 1024
        ),
    )(x, Win0, Wout0, Win1, Wout1, Win2, Wout2)
